# Initial kernel scaffold; baseline (speedup 1.0000x reference)
import jax
import jax.numpy as jnp
from jax.experimental import pallas as pl


def kernel(attrs):
    raise NotImplementedError("write your pallas kernel here")


# speedup vs baseline: 1.1689x; 1.1689x over previous
import functools

import jax
import jax.numpy as jnp
from jax.experimental import pallas as pl


def _avg_block(in_ref, out_ref, *, inv_n):
    out_ref[...] = jnp.sum(in_ref[...], axis=0) * inv_n


@functools.partial(jax.jit, static_argnames=("block_m",))
def _avg(attrs, block_m=1000):
    n, m, d = attrs.shape
    grid = (pl.cdiv(m, block_m),)
    return pl.pallas_call(
        functools.partial(_avg_block, inv_n=1.0 / n),
        grid=grid,
        in_specs=[pl.BlockSpec((n, block_m, d), lambda i: (0, i, 0))],
        out_specs=pl.BlockSpec((block_m, d), lambda i: (i, 0)),
        out_shape=jax.ShapeDtypeStruct((m, d), attrs.dtype),
    )(attrs)


def kernel(attrs):
    return _avg(attrs)
